# Initial kernel scaffold; baseline (speedup 1.0000x reference)
import jax
import jax.numpy as jnp
from jax.experimental import pallas as pl


def kernel(seq, embed, W1, b1, W2, b2, gamma, beta, wg_W, wg_b, dm_W, dm_b, q_W, q_b, out_W, out_b):
    raise NotImplementedError("write your pallas kernel here")


# speedup vs baseline: 12.1315x; 12.1315x over previous
import functools

import jax
import jax.numpy as jnp
from jax import lax
from jax.experimental import pallas as pl
from jax.experimental.pallas import tpu as pltpu
from jax.experimental.pallas import tpu_sc as plsc

V, H, FS, SS = 100000, 256, 64, 128
B, T = 64, 200
NSTEP = T - 3
THRESHOLD = 0.3
NW = 32


def _sc_gather_body(n_per_w, chunks, table_hbm, idx_hbm, out_hbm, idx_v, rows_v, sem):
    wid = lax.axis_index("s") * 2 + lax.axis_index("c")
    base = wid * n_per_w
    pltpu.sync_copy(idx_hbm.at[pl.ds(base, n_per_w)], idx_v)
    copies = []
    off = 0
    for c in chunks:
        copies.append(pltpu.async_copy(
            table_hbm.at[idx_v.at[pl.ds(off, c)]],
            rows_v.at[pl.ds(off, c)], sem))
        off += c
    for cp in copies:
        cp.wait()
    pltpu.sync_copy(rows_v, out_hbm.at[pl.ds(base, n_per_w)])


def _gather_rows(table, idx):
    n = idx.shape[0]
    d = table.shape[1]
    n_per_w = n // NW
    chunks = []
    left = n_per_w
    while left > 0:
        c = min(128, left)
        chunks.append(c)
        left -= c
    mesh = plsc.VectorSubcoreMesh(core_axis_name="c", subcore_axis_name="s")
    fn = pl.kernel(
        functools.partial(_sc_gather_body, n_per_w, tuple(chunks)),
        out_type=jax.ShapeDtypeStruct((n, d), jnp.float32),
        mesh=mesh,
        scratch_types=[
            pltpu.VMEM((n_per_w,), jnp.int32),
            pltpu.VMEM((n_per_w, d), jnp.float32),
            pltpu.SemaphoreType.DMA,
        ],
    )
    return fn(table, idx)


def _encoder_body(h0_ref, W1_ref, b1_ref, W2_ref, b2_ref, g_ref, be_ref,
                  wgT_ref, dmT_ref, wb_ref, db_ref, h_ref, zwd_ref):
    h0 = h0_ref[...]
    f = jnp.maximum(jnp.dot(h0, W1_ref[...], preferred_element_type=jnp.float32)
                    + b1_ref[...], 0.0)
    f = jnp.dot(f, W2_ref[...], preferred_element_type=jnp.float32) + b2_ref[...]
    x = h0 + f
    mu = jnp.mean(x, axis=-1, keepdims=True)
    var = jnp.mean((x - mu) ** 2, axis=-1, keepdims=True)
    h = (x - mu) / jnp.sqrt(var + 1e-5) * g_ref[...] + be_ref[...]
    h_ref[...] = h
    zw = jnp.sum(h * wgT_ref[...], axis=-1, keepdims=True) + wb_ref[...]
    zd = jnp.sum(h * dmT_ref[...], axis=-1, keepdims=True) + db_ref[...]
    zwd_ref[...] = jnp.concatenate([zw, zd], axis=-1)


def _encoder(h0, W1, b1, W2, b2, gamma, beta, wg_W, wg_b, dm_W, dm_b):
    n = h0.shape[0]
    blk = 512
    grid = n // blk
    full = lambda shape: pl.BlockSpec(shape, lambda i: (0, 0))
    return pl.pallas_call(
        _encoder_body,
        grid=(grid,),
        in_specs=[
            pl.BlockSpec((blk, H), lambda i: (i, 0)),
            full((H, 2 * H)), full((1, 2 * H)), full((2 * H, H)), full((1, H)),
            full((1, H)), full((1, H)),
            full((1, H)), full((1, H)), full((1, 1)), full((1, 1)),
        ],
        out_specs=[
            pl.BlockSpec((blk, H), lambda i: (i, 0)),
            pl.BlockSpec((blk, 2), lambda i: (i, 0)),
        ],
        out_shape=[
            jax.ShapeDtypeStruct((n, H), jnp.float32),
            jax.ShapeDtypeStruct((n, 2), jnp.float32),
        ],
    )(h0, W1, b1.reshape(1, -1), W2, b2.reshape(1, -1),
      gamma.reshape(1, -1), beta.reshape(1, -1),
      wg_W.reshape(1, -1), dm_W.reshape(1, -1),
      wg_b.reshape(1, 1), dm_b.reshape(1, 1))


def _writer_body(zwT_ref, dT_ref, db_ref, fgidx_ref, fmask_ref, sgidx_ref, smask_ref):
    sl_f = lax.broadcasted_iota(jnp.int32, (FS, B), 0)
    sl_s = lax.broadcasted_iota(jnp.int32, (SS, B), 0)
    BIG = jnp.int32(1 << 30)

    def first_idx(mask_i32, sl):
        return jnp.min(jnp.where(mask_i32 != 0, sl, BIG), axis=0, keepdims=True)

    def body(t, st):
        f_score, f_age, f_used, f_tok, s_age, s_used, s_tok = st
        zw_t = zwT_ref[pl.ds(t, 1)].reshape(1, B)
        act = ((1.0 / (1.0 + jnp.exp(-zw_t))) >= 0.4).astype(jnp.int32)
        d_t = dT_ref[pl.ds(t, 1)].reshape(1, B)
        f_age = f_age + f_used
        s_age = s_age + s_used
        has_free_f = jnp.min(f_used, axis=0, keepdims=True) == 0
        free_f = first_idx(1 - f_used, sl_f)
        ds_min = jnp.min(f_score, axis=0, keepdims=True)
        dem = first_idx((f_score == ds_min).astype(jnp.int32), sl_f)
        demote = (act != 0) & ~has_free_f
        dh_tok = jnp.sum(jnp.where(sl_f == dem, f_tok, 0), axis=0, keepdims=True)
        has_free_s = jnp.min(s_used, axis=0, keepdims=True) == 0
        free_s = first_idx(1 - s_used, sl_s)
        age_max = jnp.max(s_age, axis=0, keepdims=True)
        evict_s = first_idx((s_age == age_max).astype(jnp.int32), sl_s)
        ss = jnp.where(has_free_s, free_s, evict_s)
        s_write = (sl_s == ss) & demote
        s_tok = jnp.where(s_write, dh_tok, s_tok)
        s_age = jnp.where(s_write, 0, s_age)
        s_used = jnp.where(s_write, 1, s_used)
        fsel = jnp.where(has_free_f, free_f, dem)
        f_write = (sl_f == fsel) & (act != 0)
        f_score = jnp.where(f_write, d_t, f_score)
        f_tok = jnp.where(f_write, t, f_tok)
        f_age = jnp.where(f_write, 0, f_age)
        f_used = jnp.where(f_write, 1, f_used)
        return (f_score, f_age, f_used, f_tok, s_age, s_used, s_tok)

    init = (
        jnp.full((FS, B), db_ref[0, 0], jnp.float32),
        jnp.zeros((FS, B), jnp.int32),
        jnp.zeros((FS, B), jnp.int32),
        jnp.zeros((FS, B), jnp.int32),
        jnp.zeros((SS, B), jnp.int32),
        jnp.zeros((SS, B), jnp.int32),
        jnp.zeros((SS, B), jnp.int32),
    )
    f_score, f_age, f_used, f_tok, s_age, s_used, s_tok = lax.fori_loop(
        0, NSTEP, body, init)

    b_f = lax.broadcasted_iota(jnp.int32, (FS, B), 1)
    b_s = lax.broadcasted_iota(jnp.int32, (SS, B), 1)
    fgidx_ref[...] = jnp.where(f_used != 0, b_f * T + f_tok, 0)
    fmask_ref[...] = (f_used != 0).astype(jnp.float32)
    sgidx_ref[...] = jnp.where(s_used != 0, b_s * T + s_tok, 0)
    smask_ref[...] = (s_used != 0).astype(jnp.float32)


def _writer(zwT, dT, dm_b):
    return pl.pallas_call(
        _writer_body,
        out_shape=[
            jax.ShapeDtypeStruct((FS, B), jnp.int32),
            jax.ShapeDtypeStruct((FS, B), jnp.float32),
            jax.ShapeDtypeStruct((SS, B), jnp.int32),
            jax.ShapeDtypeStruct((SS, B), jnp.float32),
        ],
    )(zwT, dT, dm_b.reshape(1, 1))


def _readout_body(fmem_ref, smem_ref, fmask_ref, smask_ref, hlast_ref,
                  qW_ref, qb_ref, ctx_ref, need_ref):
    q = jnp.dot(hlast_ref[...], qW_ref[...],
                preferred_element_type=jnp.float32) + qb_ref[...]

    def scores_loop(mem_ref, nslots):
        lane = lax.broadcasted_iota(jnp.int32, (B, nslots), 1)

        def body(s, sc):
            v = mem_ref[pl.ds(s, 1)].reshape(B, H)
            contrib = jnp.sum(v * q, axis=-1, keepdims=True)
            return jnp.where(lane == s, contrib, sc)
        return lax.fori_loop(0, nslots, body, jnp.zeros((B, nslots), jnp.float32))

    def softmax(x):
        m = jnp.max(x, axis=-1, keepdims=True)
        e = jnp.exp(x - m)
        return e / jnp.sum(e, axis=-1, keepdims=True)

    fmask = fmask_ref[...]
    smask = smask_ref[...]
    f_sc = jnp.where(fmask == 0, -1e9, scores_loop(fmem_ref, FS))
    f_attn = softmax(f_sc)
    max_attn = jnp.max(f_attn, axis=-1, keepdims=True)
    s_sc = jnp.where(smask == 0, -1e9, scores_loop(smem_ref, SS))
    s_attn = softmax(s_sc)

    def ctx_loop(mem_ref, attn, mask, nslots):
        lane = lax.broadcasted_iota(jnp.int32, (B, nslots), 1)
        wm = attn * mask

        def body(s, acc):
            v = mem_ref[pl.ds(s, 1)].reshape(B, H)
            w = jnp.sum(jnp.where(lane == s, wm, 0.0), axis=-1, keepdims=True)
            return acc + w * v
        return lax.fori_loop(0, nslots, body, jnp.zeros((B, H), jnp.float32))

    f_ctx = ctx_loop(fmem_ref, f_attn, fmask, FS)
    s_ctx = ctx_loop(smem_ref, s_attn, smask, SS)
    need = (max_attn < THRESHOLD).astype(jnp.float32)
    ctx_ref[...] = f_ctx + need * s_ctx
    need_ref[...] = need


def _readout(fmem, smem, fmask, smask, h_last, q_W, q_b):
    return pl.pallas_call(
        _readout_body,
        out_shape=[
            jax.ShapeDtypeStruct((B, H), jnp.float32),
            jax.ShapeDtypeStruct((B, 1), jnp.float32),
        ],
    )(fmem, smem, fmask, smask, h_last, q_W, q_b.reshape(1, -1))


def _logits_body(ctx_ref, w_ref, b_ref, out_ref):
    out_ref[...] = jnp.dot(ctx_ref[...], w_ref[...],
                           preferred_element_type=jnp.float32) + b_ref[...]


def _logits(ctx, out_W, out_b):
    vb = 4096
    grid = pl.cdiv(V, vb)
    return pl.pallas_call(
        _logits_body,
        grid=(grid,),
        in_specs=[
            pl.BlockSpec((B, H), lambda i: (0, 0)),
            pl.BlockSpec((H, vb), lambda i: (0, i)),
            pl.BlockSpec((1, vb), lambda i: (0, i)),
        ],
        out_specs=pl.BlockSpec((B, vb), lambda i: (0, i)),
        out_shape=jax.ShapeDtypeStruct((B, V), jnp.float32),
    )(ctx, out_W, out_b.reshape(1, -1))


def kernel(seq, embed, W1, b1, W2, b2, gamma, beta, wg_W, wg_b, dm_W, dm_b,
           q_W, q_b, out_W, out_b):
    seq_flat = seq.reshape(-1).astype(jnp.int32)
    h0 = _gather_rows(embed, seq_flat)
    h, zwd = _encoder(h0, W1, b1, W2, b2, gamma, beta, wg_W, wg_b, dm_W, dm_b)
    zwT = zwd[:, 0].reshape(B, T).T.reshape(T, 1, B)
    dT = zwd[:, 1].reshape(B, T).T.reshape(T, 1, B)
    fgidx, fmask, sgidx, smask = _writer(zwT, dT, dm_b)
    idx_all = jnp.concatenate([fgidx.reshape(-1), sgidx.reshape(-1)])
    mem = _gather_rows(h, idx_all)
    fmem = mem[:B * FS].reshape(FS, B, H)
    smem = mem[B * FS:].reshape(SS, B, H)
    h_last = h.reshape(B, T, H)[:, T - 1, :]
    ctx, need = _readout(fmem, smem, fmask.T, smask.T, h_last, q_W, q_b)
    logits = _logits(ctx, out_W, out_b)
    return (logits, need[:, 0])


# speedup vs baseline: 13.0086x; 1.0723x over previous
import functools

import jax
import jax.numpy as jnp
from jax import lax
from jax.experimental import pallas as pl
from jax.experimental.pallas import tpu as pltpu
from jax.experimental.pallas import tpu_sc as plsc

V, H, FS, SS = 100000, 256, 64, 128
B, T = 64, 200
NSTEP = T - 3
THRESHOLD = 0.3
NW = 32


def _sc_gather_body(n_per_w, chunks, table_hbm, idx_hbm, out_hbm, idx_v, rows_v, sem):
    wid = lax.axis_index("s") * 2 + lax.axis_index("c")
    base = wid * n_per_w
    pltpu.sync_copy(idx_hbm.at[pl.ds(base, n_per_w)], idx_v)
    copies = []
    off = 0
    for c in chunks:
        copies.append(pltpu.async_copy(
            table_hbm.at[idx_v.at[pl.ds(off, c)]],
            rows_v.at[pl.ds(off, c)], sem))
        off += c
    for cp in copies:
        cp.wait()
    pltpu.sync_copy(rows_v, out_hbm.at[pl.ds(base, n_per_w)])


def _gather_rows(table, idx):
    n = idx.shape[0]
    d = table.shape[1]
    n_per_w = n // NW
    chunks = []
    left = n_per_w
    while left > 0:
        c = min(128, left)
        chunks.append(c)
        left -= c
    mesh = plsc.VectorSubcoreMesh(core_axis_name="c", subcore_axis_name="s")
    fn = pl.kernel(
        functools.partial(_sc_gather_body, n_per_w, tuple(chunks)),
        out_type=jax.ShapeDtypeStruct((n, d), jnp.float32),
        mesh=mesh,
        scratch_types=[
            pltpu.VMEM((n_per_w,), jnp.int32),
            pltpu.VMEM((n_per_w, d), jnp.float32),
            pltpu.SemaphoreType.DMA,
        ],
    )
    return fn(table, idx)


def _encoder_body(h0_ref, W1_ref, b1_ref, W2_ref, b2_ref, g_ref, be_ref,
                  wgT_ref, dmT_ref, wb_ref, db_ref, h_ref, zwd_ref):
    h0 = h0_ref[...]
    f = jnp.maximum(jnp.dot(h0, W1_ref[...], preferred_element_type=jnp.float32)
                    + b1_ref[...], 0.0)
    f = jnp.dot(f, W2_ref[...], preferred_element_type=jnp.float32) + b2_ref[...]
    x = h0 + f
    mu = jnp.mean(x, axis=-1, keepdims=True)
    var = jnp.mean((x - mu) ** 2, axis=-1, keepdims=True)
    h = (x - mu) / jnp.sqrt(var + 1e-5) * g_ref[...] + be_ref[...]
    h_ref[...] = h
    zw = jnp.sum(h * wgT_ref[...], axis=-1, keepdims=True) + wb_ref[...]
    zd = jnp.sum(h * dmT_ref[...], axis=-1, keepdims=True) + db_ref[...]
    zwd_ref[...] = jnp.concatenate([zw, zd], axis=-1)


def _encoder(h0, W1, b1, W2, b2, gamma, beta, wg_W, wg_b, dm_W, dm_b):
    n = h0.shape[0]
    blk = 512
    grid = n // blk
    full = lambda shape: pl.BlockSpec(shape, lambda i: (0, 0))
    return pl.pallas_call(
        _encoder_body,
        grid=(grid,),
        in_specs=[
            pl.BlockSpec((blk, H), lambda i: (i, 0)),
            full((H, 2 * H)), full((1, 2 * H)), full((2 * H, H)), full((1, H)),
            full((1, H)), full((1, H)),
            full((1, H)), full((1, H)), full((1, 1)), full((1, 1)),
        ],
        out_specs=[
            pl.BlockSpec((blk, H), lambda i: (i, 0)),
            pl.BlockSpec((blk, 2), lambda i: (i, 0)),
        ],
        out_shape=[
            jax.ShapeDtypeStruct((n, H), jnp.float32),
            jax.ShapeDtypeStruct((n, 2), jnp.float32),
        ],
    )(h0, W1, b1.reshape(1, -1), W2, b2.reshape(1, -1),
      gamma.reshape(1, -1), beta.reshape(1, -1),
      wg_W.reshape(1, -1), dm_W.reshape(1, -1),
      wg_b.reshape(1, 1), dm_b.reshape(1, 1))


def _writer_body(zwT_ref, dT_ref, db_ref, fgidx_ref, fmask_ref, sgidx_ref, smask_ref):
    sl_f = lax.broadcasted_iota(jnp.int32, (FS, B), 0)
    sl_s = lax.broadcasted_iota(jnp.int32, (SS, B), 0)
    BIG = jnp.int32(1 << 30)

    def body(t, st):
        f_score, f_tok, s_age, s_tok = st
        zw_t = zwT_ref[pl.ds(t, 1)].reshape(1, B)
        act = (1.0 / (1.0 + jnp.exp(-zw_t))) >= 0.4
        d_t = dT_ref[pl.ds(t, 1)].reshape(1, B)
        ds_min = jnp.min(f_score, axis=0, keepdims=True)
        key2 = jnp.min(jnp.where(f_score == ds_min, sl_f * 1024 + f_tok + 1, BIG),
                       axis=0, keepdims=True)
        dem = key2 >> 10
        dh_tok = (key2 & 1023) - 1
        free_f = jnp.min(jnp.where(f_tok < 0, sl_f, BIG), axis=0, keepdims=True)
        has_free_f = free_f < BIG
        fsel = jnp.where(has_free_f, free_f, dem)
        demote = act & ~has_free_f
        s_age_inc = s_age + (s_age >= 0).astype(jnp.int32)
        skey = jnp.where(s_age < 0, sl_s, SS + (1024 - s_age_inc) * SS + sl_s)
        ss = jnp.min(skey, axis=0, keepdims=True) & (SS - 1)
        s_write = (sl_s == ss) & demote
        s_tok = jnp.where(s_write, dh_tok, s_tok)
        s_age = jnp.where(s_write, 0, s_age_inc)
        f_write = (sl_f == fsel) & act
        f_score = jnp.where(f_write, d_t, f_score)
        f_tok = jnp.where(f_write, t, f_tok)
        return (f_score, f_tok, s_age, s_tok)

    init = (
        jnp.full((FS, B), db_ref[0, 0], jnp.float32),
        jnp.full((FS, B), -1, jnp.int32),
        jnp.full((SS, B), -1, jnp.int32),
        jnp.zeros((SS, B), jnp.int32),
    )
    f_score, f_tok, s_age, s_tok = lax.fori_loop(0, NSTEP, body, init)

    b_f = lax.broadcasted_iota(jnp.int32, (FS, B), 1)
    b_s = lax.broadcasted_iota(jnp.int32, (SS, B), 1)
    fgidx_ref[...] = jnp.where(f_tok >= 0, b_f * T + f_tok, 0)
    fmask_ref[...] = (f_tok >= 0).astype(jnp.float32)
    sgidx_ref[...] = jnp.where(s_age >= 0, b_s * T + s_tok, 0)
    smask_ref[...] = (s_age >= 0).astype(jnp.float32)


def _writer(zwT, dT, dm_b):
    return pl.pallas_call(
        _writer_body,
        out_shape=[
            jax.ShapeDtypeStruct((FS, B), jnp.int32),
            jax.ShapeDtypeStruct((FS, B), jnp.float32),
            jax.ShapeDtypeStruct((SS, B), jnp.int32),
            jax.ShapeDtypeStruct((SS, B), jnp.float32),
        ],
    )(zwT, dT, dm_b.reshape(1, 1))


def _readout_body(mem_ref, fmask_ref, smask_ref, hlast_ref,
                  qW_ref, qb_ref, ctx_ref, need_ref):
    q = jnp.dot(hlast_ref[...], qW_ref[...],
                preferred_element_type=jnp.float32) + qb_ref[...]

    def scores_loop(base, nslots):
        lane = lax.broadcasted_iota(jnp.int32, (B, nslots), 1)

        def body(s, sc):
            v = mem_ref[pl.ds(base + s, 1)].reshape(B, H)
            contrib = jnp.sum(v * q, axis=-1, keepdims=True)
            return jnp.where(lane == s, contrib, sc)
        return lax.fori_loop(0, nslots, body, jnp.zeros((B, nslots), jnp.float32))

    def softmax(x):
        m = jnp.max(x, axis=-1, keepdims=True)
        e = jnp.exp(x - m)
        return e / jnp.sum(e, axis=-1, keepdims=True)

    fmask = fmask_ref[...]
    smask = smask_ref[...]
    f_sc = jnp.where(fmask == 0, -1e9, scores_loop(0, FS))
    f_attn = softmax(f_sc)
    max_attn = jnp.max(f_attn, axis=-1, keepdims=True)
    s_sc = jnp.where(smask == 0, -1e9, scores_loop(FS, SS))
    s_attn = softmax(s_sc)

    def ctx_loop(base, attn, mask, nslots):
        lane = lax.broadcasted_iota(jnp.int32, (B, nslots), 1)
        wm = attn * mask

        def body(s, acc):
            v = mem_ref[pl.ds(base + s, 1)].reshape(B, H)
            w = jnp.sum(jnp.where(lane == s, wm, 0.0), axis=-1, keepdims=True)
            return acc + w * v
        return lax.fori_loop(0, nslots, body, jnp.zeros((B, H), jnp.float32))

    f_ctx = ctx_loop(0, f_attn, fmask, FS)
    s_ctx = ctx_loop(FS, s_attn, smask, SS)
    need = (max_attn < THRESHOLD).astype(jnp.float32)
    ctx_ref[...] = f_ctx + need * s_ctx
    need_ref[...] = need


def _readout(mem3, fmask, smask, h_last, q_W, q_b):
    return pl.pallas_call(
        _readout_body,
        out_shape=[
            jax.ShapeDtypeStruct((B, H), jnp.float32),
            jax.ShapeDtypeStruct((B, 1), jnp.float32),
        ],
    )(mem3, fmask, smask, h_last, q_W, q_b.reshape(1, -1))


def _logits_body(ctx_ref, w_ref, b_ref, out_ref):
    out_ref[...] = jnp.dot(ctx_ref[...], w_ref[...],
                           preferred_element_type=jnp.float32) + b_ref[...]


def _logits(ctx, out_W, out_b):
    vb = 4096
    grid = pl.cdiv(V, vb)
    return pl.pallas_call(
        _logits_body,
        grid=(grid,),
        in_specs=[
            pl.BlockSpec((B, H), lambda i: (0, 0)),
            pl.BlockSpec((H, vb), lambda i: (0, i)),
            pl.BlockSpec((1, vb), lambda i: (0, i)),
        ],
        out_specs=pl.BlockSpec((B, vb), lambda i: (0, i)),
        out_shape=jax.ShapeDtypeStruct((B, V), jnp.float32),
    )(ctx, out_W, out_b.reshape(1, -1))


def kernel(seq, embed, W1, b1, W2, b2, gamma, beta, wg_W, wg_b, dm_W, dm_b,
           q_W, q_b, out_W, out_b):
    seq_flat = seq.reshape(-1).astype(jnp.int32)
    h0 = _gather_rows(embed, seq_flat)
    h, zwd = _encoder(h0, W1, b1, W2, b2, gamma, beta, wg_W, wg_b, dm_W, dm_b)
    zwT = zwd[:, 0].reshape(B, T).T.reshape(T, 1, B)
    dT = zwd[:, 1].reshape(B, T).T.reshape(T, 1, B)
    fgidx, fmask, sgidx, smask = _writer(zwT, dT, dm_b)
    idx_all = jnp.concatenate([fgidx.reshape(-1), sgidx.reshape(-1)])
    mem = _gather_rows(h, idx_all)
    mem3 = mem.reshape(FS + SS, B, H)
    h_last = h.reshape(B, T, H)[:, T - 1, :]
    ctx, need = _readout(mem3, fmask.T, smask.T, h_last, q_W, q_b)
    logits = _logits(ctx, out_W, out_b)
    return (logits, need[:, 0])
